# baseline (speedup 1.0000x reference)
import functools

import jax
import jax.numpy as jnp
from jax import lax
from jax.experimental import pallas as pl
from jax.experimental.pallas import tpu as pltpu
from jax.experimental.pallas import tpu_sc as plsc

N = 10000
NP = 10240
E = 320000
D = 128
DH = D // 2
NC = 2
NS = 16
NW = NC * NS
NPT = NP // NS
DEGW = 16

EPAD = 327680

EPT_DEG = EPAD // NW
KDEG = 128
NCH_DEG = EPT_DEG // KDEG

K = 128
EPT_AGG = EPAD // NS
NCH_AGG = EPT_AGG // K

_mesh = plsc.VectorSubcoreMesh(core_axis_name="c", subcore_axis_name="s")


@functools.partial(
    pl.kernel,
    mesh=_mesh,
    out_type=(
        jax.ShapeDtypeStruct((NP, DEGW), jnp.float32),
        jax.ShapeDtypeStruct((NP, DEGW), jnp.float32),
    ),
    scratch_types=[
        pltpu.VMEM((4, KDEG), jnp.int32),
        pltpu.VMEM((KDEG, DEGW), jnp.float32),
        pltpu.VMEM((NPT, DEGW), jnp.float32),
        pltpu.VMEM_SHARED((NP, DEGW), jnp.float32),
        pltpu.SemaphoreType.DMA,
        pltpu.SemaphoreType.DMA,
        pltpu.SemaphoreType.DMA,
        pltpu.SemaphoreType.DMA,
    ],
    compiler_params=pltpu.CompilerParams(use_tc_tiling_on_sc=False),
)
def _deg_kernel(dst_hbm, d0_hbm, d1_hbm, idx_v, ones_v, zero_v, deg_sh,
                isem0, isem1, isem2, isem3):
    c = lax.axis_index("c")
    s = lax.axis_index("s")
    wid = c * NS + s
    isems = [isem0, isem1, isem2, isem3]
    ebase = wid * EPT_DEG

    def idx_fetch(j, b):
        pltpu.async_copy(dst_hbm.at[pl.ds(ebase + j * KDEG, KDEG)],
                         idx_v.at[b], isems[b])

    def idx_wait(b):
        pltpu.make_async_copy(dst_hbm.at[pl.ds(0, KDEG)], idx_v.at[b],
                              isems[b]).wait()

    def fill_ones(i, _):
        ones_v[i, :] = jnp.full((DEGW,), 1.0, jnp.float32)
        return 0

    lax.fori_loop(0, KDEG, fill_ones, 0)

    def fill_zero(i, _):
        zero_v[i, :] = jnp.zeros((DEGW,), jnp.float32)
        return 0

    lax.fori_loop(0, NPT, fill_zero, 0)

    idx_fetch(0, 0)
    idx_fetch(1, 1)
    idx_fetch(2, 2)
    idx_fetch(3, 3)
    pltpu.sync_copy(zero_v, deg_sh.at[pl.ds(s * NPT, NPT)])
    plsc.subcore_barrier()

    def body(jo, _):
        for b in range(4):
            j = 4 * jo + b
            idx_wait(b)
            pltpu.sync_copy(ones_v, deg_sh.at[idx_v.at[b]], add=True)

            @pl.when(j + 4 < NCH_DEG)
            def _():
                idx_fetch(j + 4, b)

        return 0

    lax.fori_loop(0, NCH_DEG // 4, body, 0)
    plsc.subcore_barrier()

    @pl.when(c == 0)
    def _():
        pltpu.sync_copy(deg_sh.at[pl.ds(s * NPT, NPT)],
                        d0_hbm.at[pl.ds(s * NPT, NPT)])

    @pl.when(c == 1)
    def _():
        pltpu.sync_copy(deg_sh.at[pl.ds(s * NPT, NPT)],
                        d1_hbm.at[pl.ds(s * NPT, NPT)])


@functools.partial(
    pl.kernel,
    mesh=_mesh,
    out_type=jax.ShapeDtypeStruct((2 * NP, DH), jnp.float32),
    scratch_types=[
        pltpu.VMEM((4, K), jnp.int32),
        pltpu.VMEM((4, K), jnp.int32),
        pltpu.VMEM((4, K, DH), jnp.float32),
        pltpu.VMEM_SHARED((NP, DH), jnp.float32),
        pltpu.SemaphoreType.DMA,
        pltpu.SemaphoreType.DMA,
        pltpu.SemaphoreType.DMA,
        pltpu.SemaphoreType.DMA,
        pltpu.SemaphoreType.DMA,
        pltpu.SemaphoreType.DMA,
        pltpu.SemaphoreType.DMA,
        pltpu.SemaphoreType.DMA,
    ],
    compiler_params=pltpu.CompilerParams(use_tc_tiling_on_sc=False),
)
def _agg_kernel(g_hbm, srcb_hbm, dst_hbm, p_hbm,
                sidx_v, didx_v, rows_v, acc_sh,
                gsem0, gsem1, gsem2, gsem3, isem0, isem1, isem2, isem3):
    c = lax.axis_index("c")
    s = lax.axis_index("s")
    gsems = [gsem0, gsem1, gsem2, gsem3]
    isems = [isem0, isem1, isem2, isem3]
    ebase = s * EPT_AGG

    def idx_fetch(j, b):
        base = ebase + j * K
        pltpu.async_copy(srcb_hbm.at[c, pl.ds(base, K)], sidx_v.at[b],
                         isems[b])
        pltpu.async_copy(dst_hbm.at[pl.ds(base, K)], didx_v.at[b], isems[b])

    def idx_wait(b):
        pltpu.make_async_copy(srcb_hbm.at[c, pl.ds(0, K)], sidx_v.at[b],
                              isems[b]).wait()
        pltpu.make_async_copy(dst_hbm.at[pl.ds(0, K)], didx_v.at[b],
                              isems[b]).wait()

    def gather_start(b):
        pltpu.async_copy(g_hbm.at[sidx_v.at[b]], rows_v.at[b], gsems[b])

    def gather_wait(b):
        pltpu.make_async_copy(g_hbm.at[sidx_v.at[b]], rows_v.at[b],
                              gsems[b]).wait()

    pltpu.sync_copy(g_hbm.at[pl.ds(c * NP + s * NPT, NPT)],
                    acc_sh.at[pl.ds(s * NPT, NPT)])
    plsc.subcore_barrier()

    idx_fetch(0, 0)
    idx_fetch(1, 1)
    idx_fetch(2, 2)
    idx_wait(0)
    gather_start(0)
    idx_wait(1)
    gather_start(1)
    idx_fetch(3, 3)

    def body(jo, _):
        for b in range(4):
            j = 4 * jo + b
            b2 = (b + 2) % 4
            gather_wait(b)

            @pl.when(j + 2 < NCH_AGG)
            def _():
                idx_wait(b2)
                gather_start(b2)

            pltpu.sync_copy(rows_v.at[b], acc_sh.at[didx_v.at[b]], add=True)

            @pl.when(j + 4 < NCH_AGG)
            def _():
                idx_fetch(j + 4, b)

        return 0

    lax.fori_loop(0, NCH_AGG // 4, body, 0)
    plsc.subcore_barrier()
    pltpu.sync_copy(acc_sh.at[pl.ds(s * NPT, NPT)],
                    p_hbm.at[pl.ds(c * NP + s * NPT, NPT)])


SBLK = 640
FBLK = 1000


def _mm_body(x_ref, w_ref, h_ref):
    h_ref[...] = jnp.dot(x_ref[...], w_ref[...],
                         preferred_element_type=jnp.float32)


_mm_call = pl.pallas_call(
    _mm_body,
    grid=(NP // SBLK,),
    in_specs=[
        pl.BlockSpec((SBLK, D), lambda i: (i, 0)),
        pl.BlockSpec((D, D), lambda i: (0, 0)),
    ],
    out_specs=pl.BlockSpec((SBLK, D), lambda i: (i, 0)),
    out_shape=jax.ShapeDtypeStruct((NP, D), jnp.float32),
)


def _scale_body(h_ref, d0_ref, d1_ref, g_ref):
    deg = 1.0 + d0_ref[:, 0:1] + d1_ref[:, 0:1]
    dis = lax.rsqrt(deg)
    g = h_ref[...] * dis
    g_ref[0] = g[:, :DH]
    g_ref[1] = g[:, DH:]


_scale_call = pl.pallas_call(
    _scale_body,
    grid=(NP // SBLK,),
    in_specs=[
        pl.BlockSpec((SBLK, D), lambda i: (i, 0)),
        pl.BlockSpec((SBLK, DEGW), lambda i: (i, 0)),
        pl.BlockSpec((SBLK, DEGW), lambda i: (i, 0)),
    ],
    out_specs=pl.BlockSpec((2, SBLK, DH), lambda i: (0, i, 0)),
    out_shape=jax.ShapeDtypeStruct((2, NP, DH), jnp.float32),
)


def _final_body(p0_ref, p1_ref, d0_ref, d1_ref, b_ref, o_ref):
    deg = 1.0 + d0_ref[:, 0:1] + d1_ref[:, 0:1]
    dis = lax.rsqrt(deg)
    p = jnp.concatenate([p0_ref[...], p1_ref[...]], axis=1)
    z = dis * p + b_ref[...]
    m = jnp.max(z, axis=1, keepdims=True)
    e = jnp.exp(z - m)
    ssum = jnp.sum(e, axis=1, keepdims=True)
    o_ref[...] = z - m - jnp.log(ssum)


_final_call = pl.pallas_call(
    _final_body,
    grid=(N // FBLK,),
    in_specs=[
        pl.BlockSpec((FBLK, DH), lambda i: (i, 0)),
        pl.BlockSpec((FBLK, DH), lambda i: (i, 0)),
        pl.BlockSpec((FBLK, DEGW), lambda i: (i, 0)),
        pl.BlockSpec((FBLK, DEGW), lambda i: (i, 0)),
        pl.BlockSpec((1, D), lambda i: (0, 0)),
    ],
    out_specs=pl.BlockSpec((FBLK, D), lambda i: (i, 0)),
    out_shape=jax.ShapeDtypeStruct((N, D), jnp.float32),
)


def kernel(x, edge_index, W, b):
    pad_idx = N + jnp.arange(EPAD - E, dtype=jnp.int32) % (NP - N)
    src = jnp.concatenate([edge_index[0].astype(jnp.int32), pad_idx])
    dst = jnp.concatenate([edge_index[1].astype(jnp.int32), pad_idx])
    srcb = jnp.stack([src, src + NP])
    x_p = jnp.pad(x, ((0, NP - N), (0, 0)))
    h = _mm_call(x_p, W)
    d0, d1 = _deg_kernel(dst)
    g2 = _scale_call(h, d0, d1)
    g_cat = g2.reshape(2 * NP, DH)
    p_cat = _agg_kernel(g_cat, srcb, dst)
    return _final_call(p_cat[:NP], p_cat[NP:], d0, d1, b.reshape(1, D))


# speedup vs baseline: 1.0408x; 1.0408x over previous
import functools

import jax
import jax.numpy as jnp
from jax import lax
from jax.experimental import pallas as pl
from jax.experimental.pallas import tpu as pltpu
from jax.experimental.pallas import tpu_sc as plsc

N = 10000
NP = 10240
E = 320000
D = 128
DH = D // 2
NC = 2
NS = 16
NW = NC * NS
NPT = NP // NS
DEGW = 16

EPAD = 327680

EPT_DEG = EPAD // NW
KDEG = 128
NCH_DEG = EPT_DEG // KDEG

K = 128
EPT_AGG = EPAD // NS
NCH_AGG = EPT_AGG // K

_mesh = plsc.VectorSubcoreMesh(core_axis_name="c", subcore_axis_name="s")


@functools.partial(
    pl.kernel,
    mesh=_mesh,
    out_type=(
        jax.ShapeDtypeStruct((NP, DEGW), jnp.float32),
        jax.ShapeDtypeStruct((NP, DEGW), jnp.float32),
    ),
    scratch_types=[
        pltpu.VMEM((4, KDEG), jnp.int32),
        pltpu.VMEM((KDEG, DEGW), jnp.float32),
        pltpu.VMEM((NPT, DEGW), jnp.float32),
        pltpu.VMEM_SHARED((NP, DEGW), jnp.float32),
        pltpu.SemaphoreType.DMA,
        pltpu.SemaphoreType.DMA,
        pltpu.SemaphoreType.DMA,
        pltpu.SemaphoreType.DMA,
    ],
    compiler_params=pltpu.CompilerParams(use_tc_tiling_on_sc=False),
)
def _deg_kernel(dst_hbm, d0_hbm, d1_hbm, idx_v, ones_v, zero_v, deg_sh,
                isem0, isem1, isem2, isem3):
    c = lax.axis_index("c")
    s = lax.axis_index("s")
    wid = c * NS + s
    isems = [isem0, isem1, isem2, isem3]
    ebase = wid * EPT_DEG

    def idx_fetch(j, b):
        pltpu.async_copy(dst_hbm.at[pl.ds(ebase + j * KDEG, KDEG)],
                         idx_v.at[b], isems[b])

    def idx_wait(b):
        pltpu.make_async_copy(dst_hbm.at[pl.ds(0, KDEG)], idx_v.at[b],
                              isems[b]).wait()

    def fill_ones(i, _):
        ones_v[i, :] = jnp.full((DEGW,), 1.0, jnp.float32)
        return 0

    lax.fori_loop(0, KDEG, fill_ones, 0)

    def fill_zero(i, _):
        zero_v[i, :] = jnp.zeros((DEGW,), jnp.float32)
        return 0

    lax.fori_loop(0, NPT, fill_zero, 0)

    idx_fetch(0, 0)
    idx_fetch(1, 1)
    idx_fetch(2, 2)
    idx_fetch(3, 3)
    pltpu.sync_copy(zero_v, deg_sh.at[pl.ds(s * NPT, NPT)])
    plsc.subcore_barrier()

    def body(jo, _):
        for b in range(4):
            j = 4 * jo + b
            idx_wait(b)
            pltpu.sync_copy(ones_v, deg_sh.at[idx_v.at[b]], add=True)

            @pl.when(j + 4 < NCH_DEG)
            def _():
                idx_fetch(j + 4, b)

        return 0

    lax.fori_loop(0, NCH_DEG // 4, body, 0)
    plsc.subcore_barrier()

    @pl.when(c == 0)
    def _():
        pltpu.sync_copy(deg_sh.at[pl.ds(s * NPT, NPT)],
                        d0_hbm.at[pl.ds(s * NPT, NPT)])

    @pl.when(c == 1)
    def _():
        pltpu.sync_copy(deg_sh.at[pl.ds(s * NPT, NPT)],
                        d1_hbm.at[pl.ds(s * NPT, NPT)])


@functools.partial(
    pl.kernel,
    mesh=_mesh,
    out_type=jax.ShapeDtypeStruct((2 * NP, DH), jnp.float32),
    scratch_types=[
        pltpu.VMEM((5, K), jnp.int32),
        pltpu.VMEM((5, K), jnp.int32),
        pltpu.VMEM((5, K, DH), jnp.float32),
        pltpu.VMEM_SHARED((NP, DH), jnp.float32),
        pltpu.SemaphoreType.DMA,
        pltpu.SemaphoreType.DMA,
        pltpu.SemaphoreType.DMA,
        pltpu.SemaphoreType.DMA,
        pltpu.SemaphoreType.DMA,
        pltpu.SemaphoreType.DMA,
        pltpu.SemaphoreType.DMA,
        pltpu.SemaphoreType.DMA,
        pltpu.SemaphoreType.DMA,
        pltpu.SemaphoreType.DMA,
    ],
    compiler_params=pltpu.CompilerParams(use_tc_tiling_on_sc=False),
)
def _agg_kernel(g_hbm, srcb_hbm, dst_hbm, p_hbm,
                sidx_v, didx_v, rows_v, acc_sh,
                gsem0, gsem1, gsem2, gsem3, gsem4,
                isem0, isem1, isem2, isem3, isem4):
    c = lax.axis_index("c")
    s = lax.axis_index("s")
    gsems = [gsem0, gsem1, gsem2, gsem3, gsem4]
    isems = [isem0, isem1, isem2, isem3, isem4]
    ebase = s * EPT_AGG

    def idx_fetch(j, b):
        base = ebase + j * K
        pltpu.async_copy(srcb_hbm.at[c, pl.ds(base, K)], sidx_v.at[b],
                         isems[b])
        pltpu.async_copy(dst_hbm.at[pl.ds(base, K)], didx_v.at[b], isems[b])

    def idx_wait(b):
        pltpu.make_async_copy(srcb_hbm.at[c, pl.ds(0, K)], sidx_v.at[b],
                              isems[b]).wait()
        pltpu.make_async_copy(dst_hbm.at[pl.ds(0, K)], didx_v.at[b],
                              isems[b]).wait()

    def gather_start(b):
        pltpu.async_copy(g_hbm.at[sidx_v.at[b]], rows_v.at[b], gsems[b])

    def gather_wait(b):
        pltpu.make_async_copy(g_hbm.at[sidx_v.at[b]], rows_v.at[b],
                              gsems[b]).wait()

    pltpu.sync_copy(g_hbm.at[pl.ds(c * NP + s * NPT, NPT)],
                    acc_sh.at[pl.ds(s * NPT, NPT)])
    plsc.subcore_barrier()

    idx_fetch(0, 0)
    idx_fetch(1, 1)
    idx_fetch(2, 2)
    idx_fetch(3, 3)
    idx_wait(0)
    gather_start(0)
    idx_wait(1)
    gather_start(1)
    idx_wait(2)
    gather_start(2)
    idx_fetch(4, 4)

    def body(jo, _):
        for b in range(5):
            j = 5 * jo + b
            b3 = (b + 3) % 5
            gather_wait(b)

            @pl.when(j + 3 < NCH_AGG)
            def _():
                idx_wait(b3)
                gather_start(b3)

            pltpu.sync_copy(rows_v.at[b], acc_sh.at[didx_v.at[b]], add=True)

            @pl.when(j + 5 < NCH_AGG)
            def _():
                idx_fetch(j + 5, b)

        return 0

    lax.fori_loop(0, NCH_AGG // 5, body, 0)
    plsc.subcore_barrier()
    pltpu.sync_copy(acc_sh.at[pl.ds(s * NPT, NPT)],
                    p_hbm.at[pl.ds(c * NP + s * NPT, NPT)])


SBLK = 640
FBLK = 1000


def _scale_body(x_ref, w_ref, d0_ref, d1_ref, g_ref):
    deg = 1.0 + d0_ref[:, 0:1] + d1_ref[:, 0:1]
    dis = lax.rsqrt(deg)
    h = jnp.dot(x_ref[...], w_ref[...], preferred_element_type=jnp.float32)
    g = h * dis
    g_ref[0] = g[:, :DH]
    g_ref[1] = g[:, DH:]


_scale_call = pl.pallas_call(
    _scale_body,
    grid=(NP // SBLK,),
    in_specs=[
        pl.BlockSpec((SBLK, D), lambda i: (i, 0)),
        pl.BlockSpec((D, D), lambda i: (0, 0)),
        pl.BlockSpec((SBLK, DEGW), lambda i: (i, 0)),
        pl.BlockSpec((SBLK, DEGW), lambda i: (i, 0)),
    ],
    out_specs=pl.BlockSpec((2, SBLK, DH), lambda i: (0, i, 0)),
    out_shape=jax.ShapeDtypeStruct((2, NP, DH), jnp.float32),
)


def _final_body(p0_ref, p1_ref, d0_ref, d1_ref, b_ref, o_ref):
    deg = 1.0 + d0_ref[:, 0:1] + d1_ref[:, 0:1]
    dis = lax.rsqrt(deg)
    p = jnp.concatenate([p0_ref[...], p1_ref[...]], axis=1)
    z = dis * p + b_ref[...]
    m = jnp.max(z, axis=1, keepdims=True)
    e = jnp.exp(z - m)
    ssum = jnp.sum(e, axis=1, keepdims=True)
    o_ref[...] = z - m - jnp.log(ssum)


_final_call = pl.pallas_call(
    _final_body,
    grid=(N // FBLK,),
    in_specs=[
        pl.BlockSpec((FBLK, DH), lambda i: (i, 0)),
        pl.BlockSpec((FBLK, DH), lambda i: (i, 0)),
        pl.BlockSpec((FBLK, DEGW), lambda i: (i, 0)),
        pl.BlockSpec((FBLK, DEGW), lambda i: (i, 0)),
        pl.BlockSpec((1, D), lambda i: (0, 0)),
    ],
    out_specs=pl.BlockSpec((FBLK, D), lambda i: (i, 0)),
    out_shape=jax.ShapeDtypeStruct((N, D), jnp.float32),
)


def kernel(x, edge_index, W, b):
    pad_idx = N + jnp.arange(EPAD - E, dtype=jnp.int32) % (NP - N)
    src = jnp.concatenate([edge_index[0].astype(jnp.int32), pad_idx])
    dst = jnp.concatenate([edge_index[1].astype(jnp.int32), pad_idx])
    srcb = jnp.stack([src, src + NP])
    x_p = jnp.pad(x, ((0, NP - N), (0, 0)))
    d0, d1 = _deg_kernel(dst)
    g2 = _scale_call(x_p, W, d0, d1)
    g_cat = g2.reshape(2 * NP, DH)
    p_cat = _agg_kernel(g_cat, srcb, dst)
    return _final_call(p_cat[:NP], p_cat[NP:], d0, d1, b.reshape(1, D))
